# baseline (speedup 1.0000x reference)
import functools
import math

import jax
import jax.numpy as jnp
from jax.experimental import pallas as pl
from jax.experimental.pallas import tpu as pltpu

B = 1
T = 2048
D = 2048
H = 32
DH = D // H
E = 8
DFF = 512
RATIO = 0.5

_dot = functools.partial(jax.lax.dot_general, preferred_element_type=jnp.float32)


def _matmul(a, b, dims=(((1,), (0,)), ((), ()))):
    return _dot(a, b, dims)


def _mm_body(a_ref, w_ref, b_ref, o_ref):
    o_ref[...] = _matmul(a_ref[...], w_ref[...]) + b_ref[...]


def _matmul_bias(a, w, b, bn):
    k, n = w.shape
    return pl.pallas_call(
        _mm_body,
        grid=(n // bn,),
        in_specs=[
            pl.BlockSpec((a.shape[0], k), lambda j: (0, 0)),
            pl.BlockSpec((k, bn), lambda j: (0, j)),
            pl.BlockSpec((1, bn), lambda j: (0, j)),
        ],
        out_specs=pl.BlockSpec((a.shape[0], bn), lambda j: (0, j)),
        out_shape=jax.ShapeDtypeStruct((a.shape[0], n), jnp.float32),
    )(a, w, b.reshape(1, n))


def _qkv_body(x_ref, w_ref, b_ref, sw_ref, sb_ref, sc_ref, o_ref, aw_ref):
    o_ref[...] = _matmul(x_ref[...], w_ref[...]) + b_ref[...]

    @pl.when(pl.program_id(0) == 0)
    def _aware():
        xm = jnp.sum(x_ref[...], axis=0, keepdims=True) * (1.0 / T)
        a = _dot(sw_ref[...], xm, (((0,), (1,)), ((), ())))
        a = (a + sb_ref[...]) * sc_ref[0, 0]
        aw_ref[...] = jnp.broadcast_to(a, (H, 128))


def _qkv_aware(x, w, b, sa_w, sa_b, scale, bn=512):
    k, n = w.shape
    return pl.pallas_call(
        _qkv_body,
        grid=(n // bn,),
        in_specs=[
            pl.BlockSpec((T, k), lambda j: (0, 0)),
            pl.BlockSpec((k, bn), lambda j: (0, j)),
            pl.BlockSpec((1, bn), lambda j: (0, j)),
            pl.BlockSpec((D, H), lambda j: (0, 0)),
            pl.BlockSpec((H, 1), lambda j: (0, 0)),
            pl.BlockSpec((1, 1), lambda j: (0, 0)),
        ],
        out_specs=[
            pl.BlockSpec((T, bn), lambda j: (0, j)),
            pl.BlockSpec((H, 128), lambda j: (0, 0)),
        ],
        out_shape=[
            jax.ShapeDtypeStruct((T, n), jnp.float32),
            jax.ShapeDtypeStruct((H, 128), jnp.float32),
        ],
    )(x, w, b.reshape(1, n), sa_w, sa_b.reshape(H, 1), scale.reshape(1, 1))


TQ = 1024
HP = H // 2


def _flash_body(aw_ref, q_ref, k_ref, v_ref, *args):
    if len(args) == 2:
        ex_ref, o_ref = args
    else:
        ex_ref = None
        (o_ref,) = args
    ca = aw_ref[0, 0:1, 0:1]
    cb = aw_ref[0, 1:2, 0:1]
    k2 = k_ref[...]
    v2 = v_ref[...]
    CH = 128
    lo_c = jax.lax.broadcasted_iota(jnp.int32, (CH, 128), 1) < DH
    for c in range(TQ // CH):
        sl = slice(c * CH, (c + 1) * CH)
        qc = q_ref[sl, :]
        qa = jnp.where(lo_c, qc, 0.0)
        qb = jnp.where(lo_c, 0.0, qc)
        sa = _dot(qa, k2, (((1,), (1,)), ((), ()))) * ca
        pa = jnp.exp(sa - jnp.max(sa, axis=1, keepdims=True))
        la = jnp.sum(pa, axis=1, keepdims=True)
        pva = _matmul(pa, v2)

        sb = _dot(qb, k2, (((1,), (1,)), ((), ()))) * cb
        pb = jnp.exp(sb - jnp.max(sb, axis=1, keepdims=True))
        lb = jnp.sum(pb, axis=1, keepdims=True)
        pvb = _matmul(pb, v2)

        o = jnp.where(lo_c, pva / la, pvb / lb)
        if ex_ref is not None:
            o = o + 0.3 * ex_ref[sl, :]
        o_ref[sl, :] = o * jax.nn.sigmoid(o)


def _flash_attention(qkv, aware, extra=None):
    in_specs = [
        pl.BlockSpec((1, 2, 128), lambda h, iq: (h, 0, 0)),
        pl.BlockSpec((TQ, 128), lambda h, iq: (iq, h)),
        pl.BlockSpec((T, 128), lambda h, iq: (0, HP + h)),
        pl.BlockSpec((T, 128), lambda h, iq: (0, 2 * HP + h)),
    ]
    inputs = [aware.reshape(HP, 2, 128), qkv, qkv, qkv]
    if extra is not None:
        in_specs.append(pl.BlockSpec((TQ, 128), lambda h, iq: (iq, h)))
        inputs.append(extra)
    return pl.pallas_call(
        _flash_body,
        grid=(HP, T // TQ),
        in_specs=in_specs,
        out_specs=pl.BlockSpec((TQ, 128), lambda h, iq: (iq, h)),
        out_shape=jax.ShapeDtypeStruct((T, D), jnp.float32),
    )(*inputs)


TM = 256


def _top2_combine(p):
    iota = jax.lax.broadcasted_iota(jnp.int32, p.shape, 1)
    m1 = jnp.max(p, axis=1, keepdims=True)
    i1 = jnp.min(jnp.where(p == m1, iota, E), axis=1, keepdims=True)
    oh1 = iota == i1
    pm = jnp.where(oh1, -1.0, p)
    m2 = jnp.max(pm, axis=1, keepdims=True)
    i2 = jnp.min(jnp.where(pm == m2, iota, E), axis=1, keepdims=True)
    oh2 = iota == i2
    denom = m1 + m2
    return (jnp.where(oh1, m1, 0.0) + jnp.where(oh2, m2, 0.0)) / denom


def _merge_body(h0_ref, h1_ref, w0_ref, w1_ref, b_ref, x_ref, g_ref, bb_ref,
                gw_ref, gb_ref, o_ref, c_ref):
    z = _matmul(h0_ref[...], w0_ref[...])
    z = z + _matmul(h1_ref[...], w1_ref[...])
    z = z + b_ref[...] + x_ref[...]
    mu = jnp.mean(z, axis=1, keepdims=True)
    zc = z - mu
    var = jnp.mean(zc * zc, axis=1, keepdims=True)
    xn = zc * jax.lax.rsqrt(var + 1e-5) * g_ref[...] + bb_ref[...]
    o_ref[...] = xn
    logits = _matmul(xn, gw_ref[...]) + gb_ref[...]
    ex = jnp.exp(logits - jnp.max(logits, axis=1, keepdims=True))
    p = ex / jnp.sum(ex, axis=1, keepdims=True)
    c_ref[...] = _top2_combine(p)


def _merger_ln_gate(h0, h1, w0, w1, mb, xf, g, b, gw, gb):
    return pl.pallas_call(
        _merge_body,
        grid=(T // TM,),
        in_specs=[
            pl.BlockSpec((TM, D), lambda i: (i, 0)),
            pl.BlockSpec((TM, D), lambda i: (i, 0)),
            pl.BlockSpec((D, D), lambda i: (0, 0)),
            pl.BlockSpec((D, D), lambda i: (0, 0)),
            pl.BlockSpec((1, D), lambda i: (0, 0)),
            pl.BlockSpec((TM, D), lambda i: (i, 0)),
            pl.BlockSpec((1, D), lambda i: (0, 0)),
            pl.BlockSpec((1, D), lambda i: (0, 0)),
            pl.BlockSpec((D, E), lambda i: (0, 0)),
            pl.BlockSpec((1, E), lambda i: (0, 0)),
        ],
        out_specs=[
            pl.BlockSpec((TM, D), lambda i: (i, 0)),
            pl.BlockSpec((TM, E), lambda i: (i, 0)),
        ],
        out_shape=[
            jax.ShapeDtypeStruct((T, D), jnp.float32),
            jax.ShapeDtypeStruct((T, E), jnp.float32),
        ],
    )(h0, h1, w0, w1, mb.reshape(1, D), xf, g.reshape(1, D), b.reshape(1, D),
      gw, gb.reshape(1, E))


_TCHUNK = 512
_INV_SQRT2 = 1.0 / math.sqrt(2.0)


def _moe_body(xn_ref, comb_ref, w1_ref, b1_ref, w2_ref, b2_ref, o_ref):
    e = pl.program_id(0)
    iota = jax.lax.broadcasted_iota(jnp.int32, (T, E), 1)
    c = jnp.sum(jnp.where(iota == e, comb_ref[...], 0.0), axis=1,
                keepdims=True)
    for i in range(T // _TCHUNK):
        sl = pl.ds(i * _TCHUNK, _TCHUNK)
        xb = xn_ref[sl, :]
        h = _matmul(xb, w1_ref[0]) + b1_ref[0]
        h = 0.5 * h * (1.0 + jax.lax.erf(h * _INV_SQRT2))
        y = _matmul(h, w2_ref[0]) + b2_ref[0]
        contrib = (RATIO * c[i * _TCHUNK:(i + 1) * _TCHUNK, :]) * y

        @pl.when(e == 0)
        def _first():
            o_ref[sl, :] = xb + contrib

        @pl.when(e > 0)
        def _rest():
            o_ref[sl, :] = o_ref[sl, :] + contrib


def _moe(xn, comb, e_w1, e_b1, e_w2, e_b2):
    return pl.pallas_call(
        _moe_body,
        grid=(E,),
        in_specs=[
            pl.BlockSpec((T, D), lambda e: (0, 0)),
            pl.BlockSpec((T, E), lambda e: (0, 0)),
            pl.BlockSpec((1, D, DFF), lambda e: (e, 0, 0)),
            pl.BlockSpec((1, 1, DFF), lambda e: (e, 0, 0)),
            pl.BlockSpec((1, DFF, D), lambda e: (e, 0, 0)),
            pl.BlockSpec((1, 1, D), lambda e: (e, 0, 0)),
        ],
        out_specs=pl.BlockSpec((T, D), lambda e: (0, 0)),
        out_shape=jax.ShapeDtypeStruct((T, D), jnp.float32),
    )(xn, comb, e_w1, e_b1.reshape(E, 1, DFF), e_w2, e_b2.reshape(E, 1, D))


def kernel(x, qkv_w0, qkv_b0, sa_w0, sa_b0, scale0, qkv_w1, qkv_b1, sa_w1,
           sa_b1, scale1, merger_w, merger_b, ln_g, ln_b, gate_w, gate_b,
           e_w1, e_b1, e_w2, e_b2):
    xf = x[0]

    qkv0, aw0 = _qkv_aware(xf, qkv_w0, qkv_b0, sa_w0, sa_b0, scale0)
    h0 = _flash_attention(qkv0, aw0)

    qkv1, aw1 = _qkv_aware(h0, qkv_w1, qkv_b1, sa_w1, sa_b1, scale1)
    prior_v = _matmul_bias(h0, qkv1[:, 2 * D:], jnp.zeros((D,), jnp.float32),
                           512)
    h1 = _flash_attention(qkv1, aw1, extra=prior_v)

    xn, comb = _merger_ln_gate(h0, h1, merger_w[:D], merger_w[D:], merger_b,
                               xf, ln_g, ln_b, gate_w, gate_b)
    out = _moe(xn, comb, e_w1, e_b1, e_w2, e_b2)
    return out.reshape(1, T, D)


# speedup vs baseline: 1.1492x; 1.1492x over previous
import functools
import math

import jax
import jax.numpy as jnp
from jax.experimental import pallas as pl
from jax.experimental.pallas import tpu as pltpu

B = 1
T = 2048
D = 2048
H = 32
DH = D // H
E = 8
DFF = 512
RATIO = 0.5

_dot = functools.partial(jax.lax.dot_general, preferred_element_type=jnp.float32)


def _matmul(a, b, dims=(((1,), (0,)), ((), ()))):
    return _dot(a, b, dims)


def _mm_body(a_ref, w_ref, b_ref, o_ref):
    o_ref[...] = _matmul(a_ref[...], w_ref[...]) + b_ref[...]


def _matmul_bias(a, w, b, bn):
    k, n = w.shape
    return pl.pallas_call(
        _mm_body,
        grid=(n // bn,),
        in_specs=[
            pl.BlockSpec((a.shape[0], k), lambda j: (0, 0)),
            pl.BlockSpec((k, bn), lambda j: (0, j)),
            pl.BlockSpec((1, bn), lambda j: (0, j)),
        ],
        out_specs=pl.BlockSpec((a.shape[0], bn), lambda j: (0, j)),
        out_shape=jax.ShapeDtypeStruct((a.shape[0], n), jnp.float32),
    )(a, w, b.reshape(1, n))


def _qkv_body(x_ref, w_ref, b_ref, sw_ref, sb_ref, sc_ref, o_ref, aw_ref):
    o_ref[...] = _matmul(x_ref[...], w_ref[...]) + b_ref[...]

    @pl.when(pl.program_id(0) == 0)
    def _aware():
        xm = jnp.sum(x_ref[...], axis=0, keepdims=True) * (1.0 / T)
        a = _dot(sw_ref[...], xm, (((0,), (1,)), ((), ())))
        a = (a + sb_ref[...]) * sc_ref[0, 0]
        aw_ref[...] = jnp.broadcast_to(a, (H, 128))


def _qkv_aware(x, w, b, sa_w, sa_b, scale, bn=512):
    k, n = w.shape
    return pl.pallas_call(
        _qkv_body,
        grid=(n // bn,),
        in_specs=[
            pl.BlockSpec((T, k), lambda j: (0, 0)),
            pl.BlockSpec((k, bn), lambda j: (0, j)),
            pl.BlockSpec((1, bn), lambda j: (0, j)),
            pl.BlockSpec((D, H), lambda j: (0, 0)),
            pl.BlockSpec((H, 1), lambda j: (0, 0)),
            pl.BlockSpec((1, 1), lambda j: (0, 0)),
        ],
        out_specs=[
            pl.BlockSpec((T, bn), lambda j: (0, j)),
            pl.BlockSpec((H, 128), lambda j: (0, 0)),
        ],
        out_shape=[
            jax.ShapeDtypeStruct((T, n), jnp.float32),
            jax.ShapeDtypeStruct((H, 128), jnp.float32),
        ],
    )(x, w, b.reshape(1, n), sa_w, sa_b.reshape(H, 1), scale.reshape(1, 1))


TQ = 1024
HP = H // 2


def _flash_body(aw_ref, q_ref, k_ref, v_ref, *args):
    if len(args) == 2:
        ex_ref, o_ref = args
    else:
        ex_ref = None
        (o_ref,) = args
    ca = aw_ref[0, 0:1, 0:1]
    cb = aw_ref[0, 1:2, 0:1]
    k2 = k_ref[...]
    v2 = v_ref[...]
    CH = 512
    lo_c = jax.lax.broadcasted_iota(jnp.int32, (CH, 128), 1) < DH
    for c in range(TQ // CH):
        sl = slice(c * CH, (c + 1) * CH)
        qc = q_ref[sl, :]
        qa = jnp.where(lo_c, qc, 0.0)
        qb = jnp.where(lo_c, 0.0, qc)
        sa = _dot(qa, k2, (((1,), (1,)), ((), ()))) * ca
        pa = jnp.exp(sa - jnp.max(sa, axis=1, keepdims=True))
        la = jnp.sum(pa, axis=1, keepdims=True)
        pva = _matmul(pa, v2)

        sb = _dot(qb, k2, (((1,), (1,)), ((), ()))) * cb
        pb = jnp.exp(sb - jnp.max(sb, axis=1, keepdims=True))
        lb = jnp.sum(pb, axis=1, keepdims=True)
        pvb = _matmul(pb, v2)

        o = jnp.where(lo_c, pva / la, pvb / lb)
        if ex_ref is not None:
            o = o + 0.3 * ex_ref[sl, :]
        o_ref[sl, :] = o * jax.nn.sigmoid(o)


def _flash_attention(qkv, aware, extra=None):
    in_specs = [
        pl.BlockSpec((1, 2, 128), lambda h, iq: (h, 0, 0)),
        pl.BlockSpec((TQ, 128), lambda h, iq: (iq, h)),
        pl.BlockSpec((T, 128), lambda h, iq: (0, HP + h)),
        pl.BlockSpec((T, 128), lambda h, iq: (0, 2 * HP + h)),
    ]
    inputs = [aware.reshape(HP, 2, 128), qkv, qkv, qkv]
    if extra is not None:
        in_specs.append(pl.BlockSpec((TQ, 128), lambda h, iq: (iq, h)))
        inputs.append(extra)
    return pl.pallas_call(
        _flash_body,
        grid=(HP, T // TQ),
        in_specs=in_specs,
        out_specs=pl.BlockSpec((TQ, 128), lambda h, iq: (iq, h)),
        out_shape=jax.ShapeDtypeStruct((T, D), jnp.float32),
    )(*inputs)


TM = 256


def _top2_combine(p):
    iota = jax.lax.broadcasted_iota(jnp.int32, p.shape, 1)
    m1 = jnp.max(p, axis=1, keepdims=True)
    i1 = jnp.min(jnp.where(p == m1, iota, E), axis=1, keepdims=True)
    oh1 = iota == i1
    pm = jnp.where(oh1, -1.0, p)
    m2 = jnp.max(pm, axis=1, keepdims=True)
    i2 = jnp.min(jnp.where(pm == m2, iota, E), axis=1, keepdims=True)
    oh2 = iota == i2
    denom = m1 + m2
    return (jnp.where(oh1, m1, 0.0) + jnp.where(oh2, m2, 0.0)) / denom


def _merge_body(h0_ref, h1_ref, w0_ref, w1_ref, b_ref, x_ref, g_ref, bb_ref,
                gw_ref, gb_ref, o_ref, c_ref):
    z = _matmul(h0_ref[...], w0_ref[...])
    z = z + _matmul(h1_ref[...], w1_ref[...])
    z = z + b_ref[...] + x_ref[...]
    mu = jnp.mean(z, axis=1, keepdims=True)
    zc = z - mu
    var = jnp.mean(zc * zc, axis=1, keepdims=True)
    xn = zc * jax.lax.rsqrt(var + 1e-5) * g_ref[...] + bb_ref[...]
    o_ref[...] = xn
    logits = _matmul(xn, gw_ref[...]) + gb_ref[...]
    ex = jnp.exp(logits - jnp.max(logits, axis=1, keepdims=True))
    p = ex / jnp.sum(ex, axis=1, keepdims=True)
    c_ref[...] = _top2_combine(p)


def _merger_ln_gate(h0, h1, w0, w1, mb, xf, g, b, gw, gb):
    return pl.pallas_call(
        _merge_body,
        grid=(T // TM,),
        in_specs=[
            pl.BlockSpec((TM, D), lambda i: (i, 0)),
            pl.BlockSpec((TM, D), lambda i: (i, 0)),
            pl.BlockSpec((D, D), lambda i: (0, 0)),
            pl.BlockSpec((D, D), lambda i: (0, 0)),
            pl.BlockSpec((1, D), lambda i: (0, 0)),
            pl.BlockSpec((TM, D), lambda i: (i, 0)),
            pl.BlockSpec((1, D), lambda i: (0, 0)),
            pl.BlockSpec((1, D), lambda i: (0, 0)),
            pl.BlockSpec((D, E), lambda i: (0, 0)),
            pl.BlockSpec((1, E), lambda i: (0, 0)),
        ],
        out_specs=[
            pl.BlockSpec((TM, D), lambda i: (i, 0)),
            pl.BlockSpec((TM, E), lambda i: (i, 0)),
        ],
        out_shape=[
            jax.ShapeDtypeStruct((T, D), jnp.float32),
            jax.ShapeDtypeStruct((T, E), jnp.float32),
        ],
    )(h0, h1, w0, w1, mb.reshape(1, D), xf, g.reshape(1, D), b.reshape(1, D),
      gw, gb.reshape(1, E))


_TCHUNK = 512
_INV_SQRT2 = 1.0 / math.sqrt(2.0)


def _moe_body(xn_ref, comb_ref, w1_ref, b1_ref, w2_ref, b2_ref, o_ref):
    e = pl.program_id(0)
    iota = jax.lax.broadcasted_iota(jnp.int32, (T, E), 1)
    c = jnp.sum(jnp.where(iota == e, comb_ref[...], 0.0), axis=1,
                keepdims=True)
    for i in range(T // _TCHUNK):
        sl = pl.ds(i * _TCHUNK, _TCHUNK)
        xb = xn_ref[sl, :]
        h = _matmul(xb, w1_ref[0]) + b1_ref[0]
        h = 0.5 * h * (1.0 + jax.lax.erf(h * _INV_SQRT2))
        y = _matmul(h, w2_ref[0]) + b2_ref[0]
        contrib = (RATIO * c[i * _TCHUNK:(i + 1) * _TCHUNK, :]) * y

        @pl.when(e == 0)
        def _first():
            o_ref[sl, :] = xb + contrib

        @pl.when(e > 0)
        def _rest():
            o_ref[sl, :] = o_ref[sl, :] + contrib


def _moe(xn, comb, e_w1, e_b1, e_w2, e_b2):
    return pl.pallas_call(
        _moe_body,
        grid=(E,),
        in_specs=[
            pl.BlockSpec((T, D), lambda e: (0, 0)),
            pl.BlockSpec((T, E), lambda e: (0, 0)),
            pl.BlockSpec((1, D, DFF), lambda e: (e, 0, 0)),
            pl.BlockSpec((1, 1, DFF), lambda e: (e, 0, 0)),
            pl.BlockSpec((1, DFF, D), lambda e: (e, 0, 0)),
            pl.BlockSpec((1, 1, D), lambda e: (e, 0, 0)),
        ],
        out_specs=pl.BlockSpec((T, D), lambda e: (0, 0)),
        out_shape=jax.ShapeDtypeStruct((T, D), jnp.float32),
    )(xn, comb, e_w1, e_b1.reshape(E, 1, DFF), e_w2, e_b2.reshape(E, 1, D))


def kernel(x, qkv_w0, qkv_b0, sa_w0, sa_b0, scale0, qkv_w1, qkv_b1, sa_w1,
           sa_b1, scale1, merger_w, merger_b, ln_g, ln_b, gate_w, gate_b,
           e_w1, e_b1, e_w2, e_b2):
    xf = x[0]

    qkv0, aw0 = _qkv_aware(xf, qkv_w0, qkv_b0, sa_w0, sa_b0, scale0)
    h0 = _flash_attention(qkv0, aw0)

    qkv1, aw1 = _qkv_aware(h0, qkv_w1, qkv_b1, sa_w1, sa_b1, scale1)
    prior_v = _matmul_bias(h0, qkv1[:, 2 * D:], jnp.zeros((D,), jnp.float32),
                           512)
    h1 = _flash_attention(qkv1, aw1, extra=prior_v)

    xn, comb = _merger_ln_gate(h0, h1, merger_w[:D], merger_w[D:], merger_b,
                               xf, ln_g, ln_b, gate_w, gate_b)
    out = _moe(xn, comb, e_w1, e_b1, e_w2, e_b2)
    return out.reshape(1, T, D)


# speedup vs baseline: 1.2539x; 1.0911x over previous
import functools
import math

import jax
import jax.numpy as jnp
from jax.experimental import pallas as pl
from jax.experimental.pallas import tpu as pltpu

B = 1
T = 2048
D = 2048
H = 32
DH = D // H
E = 8
DFF = 512
RATIO = 0.5

_dot = functools.partial(jax.lax.dot_general, preferred_element_type=jnp.float32)


def _matmul(a, b, dims=(((1,), (0,)), ((), ()))):
    return _dot(a, b, dims)


def _mm_body(a_ref, w_ref, b_ref, o_ref):
    o_ref[...] = _matmul(a_ref[...], w_ref[...]) + b_ref[...]


def _matmul_bias(a, w, b, bn):
    k, n = w.shape
    return pl.pallas_call(
        _mm_body,
        grid=(n // bn,),
        in_specs=[
            pl.BlockSpec((a.shape[0], k), lambda j: (0, 0)),
            pl.BlockSpec((k, bn), lambda j: (0, j)),
            pl.BlockSpec((1, bn), lambda j: (0, j)),
        ],
        out_specs=pl.BlockSpec((a.shape[0], bn), lambda j: (0, j)),
        out_shape=jax.ShapeDtypeStruct((a.shape[0], n), jnp.float32),
    )(a, w, b.reshape(1, n))


def _qkv_body(x_ref, w_ref, b_ref, sw_ref, sb_ref, sc_ref, o_ref, aw_ref):
    o_ref[...] = _matmul(x_ref[...], w_ref[...]) + b_ref[...]

    @pl.when(pl.program_id(0) == 0)
    def _aware():
        xm = jnp.sum(x_ref[...], axis=0, keepdims=True) * (1.0 / T)
        a = _dot(sw_ref[...], xm, (((0,), (1,)), ((), ())))
        a = (a + sb_ref[...]) * sc_ref[0, 0]
        aw_ref[...] = jnp.broadcast_to(a, (H, 128))


def _qkv_aware(x, w, b, sa_w, sa_b, scale, bn=512):
    k, n = w.shape
    return pl.pallas_call(
        _qkv_body,
        grid=(n // bn,),
        in_specs=[
            pl.BlockSpec((T, k), lambda j: (0, 0)),
            pl.BlockSpec((k, bn), lambda j: (0, j)),
            pl.BlockSpec((1, bn), lambda j: (0, j)),
            pl.BlockSpec((D, H), lambda j: (0, 0)),
            pl.BlockSpec((H, 1), lambda j: (0, 0)),
            pl.BlockSpec((1, 1), lambda j: (0, 0)),
        ],
        out_specs=[
            pl.BlockSpec((T, bn), lambda j: (0, j)),
            pl.BlockSpec((H, 128), lambda j: (0, 0)),
        ],
        out_shape=[
            jax.ShapeDtypeStruct((T, n), jnp.float32),
            jax.ShapeDtypeStruct((H, 128), jnp.float32),
        ],
    )(x, w, b.reshape(1, n), sa_w, sa_b.reshape(H, 1), scale.reshape(1, 1))


TQ = 2048
HP = H // 2


def _flash_body(aw_ref, q_ref, k_ref, v_ref, *args):
    if len(args) == 2:
        ex_ref, o_ref = args
    else:
        ex_ref = None
        (o_ref,) = args
    ca = aw_ref[0, 0:1, 0:1]
    cb = aw_ref[0, 1:2, 0:1]
    k2 = k_ref[...]
    v2 = v_ref[...]
    CH = 256
    lo_c = jax.lax.broadcasted_iota(jnp.int32, (CH, 128), 1) < DH
    for c in range(TQ // CH):
        sl = slice(c * CH, (c + 1) * CH)
        qc = q_ref[sl, :]
        qa = jnp.where(lo_c, qc, 0.0)
        qb = jnp.where(lo_c, 0.0, qc)
        sa = _dot(qa, k2, (((1,), (1,)), ((), ()))) * ca
        pa = jnp.exp(sa - jnp.max(sa, axis=1, keepdims=True))
        la = jnp.sum(pa, axis=1, keepdims=True)
        pva = _matmul(pa, v2)

        sb = _dot(qb, k2, (((1,), (1,)), ((), ()))) * cb
        pb = jnp.exp(sb - jnp.max(sb, axis=1, keepdims=True))
        lb = jnp.sum(pb, axis=1, keepdims=True)
        pvb = _matmul(pb, v2)

        o = jnp.where(lo_c, pva / la, pvb / lb)
        if ex_ref is not None:
            o = o + 0.3 * ex_ref[sl, :]
        o_ref[sl, :] = o * jax.nn.sigmoid(o)


def _flash_attention(qkv, aware, extra=None):
    in_specs = [
        pl.BlockSpec((1, 2, 128), lambda h, iq: (h, 0, 0)),
        pl.BlockSpec((TQ, 128), lambda h, iq: (iq, h)),
        pl.BlockSpec((T, 128), lambda h, iq: (0, HP + h)),
        pl.BlockSpec((T, 128), lambda h, iq: (0, 2 * HP + h)),
    ]
    inputs = [aware.reshape(HP, 2, 128), qkv, qkv, qkv]
    if extra is not None:
        in_specs.append(pl.BlockSpec((TQ, 128), lambda h, iq: (iq, h)))
        inputs.append(extra)
    return pl.pallas_call(
        _flash_body,
        grid=(HP, T // TQ),
        in_specs=in_specs,
        out_specs=pl.BlockSpec((TQ, 128), lambda h, iq: (iq, h)),
        out_shape=jax.ShapeDtypeStruct((T, D), jnp.float32),
    )(*inputs)


TM = 256


def _top2_combine(p):
    iota = jax.lax.broadcasted_iota(jnp.int32, p.shape, 1)
    m1 = jnp.max(p, axis=1, keepdims=True)
    i1 = jnp.min(jnp.where(p == m1, iota, E), axis=1, keepdims=True)
    oh1 = iota == i1
    pm = jnp.where(oh1, -1.0, p)
    m2 = jnp.max(pm, axis=1, keepdims=True)
    i2 = jnp.min(jnp.where(pm == m2, iota, E), axis=1, keepdims=True)
    oh2 = iota == i2
    denom = m1 + m2
    return (jnp.where(oh1, m1, 0.0) + jnp.where(oh2, m2, 0.0)) / denom


def _merge_body(h0_ref, h1_ref, w0_ref, w1_ref, b_ref, x_ref, g_ref, bb_ref,
                gw_ref, gb_ref, o_ref, c_ref):
    z = _matmul(h0_ref[...], w0_ref[...])
    z = z + _matmul(h1_ref[...], w1_ref[...])
    z = z + b_ref[...] + x_ref[...]
    mu = jnp.mean(z, axis=1, keepdims=True)
    zc = z - mu
    var = jnp.mean(zc * zc, axis=1, keepdims=True)
    xn = zc * jax.lax.rsqrt(var + 1e-5) * g_ref[...] + bb_ref[...]
    o_ref[...] = xn
    logits = _matmul(xn, gw_ref[...]) + gb_ref[...]
    ex = jnp.exp(logits - jnp.max(logits, axis=1, keepdims=True))
    p = ex / jnp.sum(ex, axis=1, keepdims=True)
    c_ref[...] = _top2_combine(p)


def _merger_ln_gate(h0, h1, w0, w1, mb, xf, g, b, gw, gb):
    return pl.pallas_call(
        _merge_body,
        grid=(T // TM,),
        in_specs=[
            pl.BlockSpec((TM, D), lambda i: (i, 0)),
            pl.BlockSpec((TM, D), lambda i: (i, 0)),
            pl.BlockSpec((D, D), lambda i: (0, 0)),
            pl.BlockSpec((D, D), lambda i: (0, 0)),
            pl.BlockSpec((1, D), lambda i: (0, 0)),
            pl.BlockSpec((TM, D), lambda i: (i, 0)),
            pl.BlockSpec((1, D), lambda i: (0, 0)),
            pl.BlockSpec((1, D), lambda i: (0, 0)),
            pl.BlockSpec((D, E), lambda i: (0, 0)),
            pl.BlockSpec((1, E), lambda i: (0, 0)),
        ],
        out_specs=[
            pl.BlockSpec((TM, D), lambda i: (i, 0)),
            pl.BlockSpec((TM, E), lambda i: (i, 0)),
        ],
        out_shape=[
            jax.ShapeDtypeStruct((T, D), jnp.float32),
            jax.ShapeDtypeStruct((T, E), jnp.float32),
        ],
    )(h0, h1, w0, w1, mb.reshape(1, D), xf, g.reshape(1, D), b.reshape(1, D),
      gw, gb.reshape(1, E))


_TCHUNK = 512
_INV_SQRT2 = 1.0 / math.sqrt(2.0)


def _moe_body(xn_ref, comb_ref, w1_ref, b1_ref, w2_ref, b2_ref, o_ref):
    e = pl.program_id(0)
    iota = jax.lax.broadcasted_iota(jnp.int32, (T, E), 1)
    c = jnp.sum(jnp.where(iota == e, comb_ref[...], 0.0), axis=1,
                keepdims=True)
    for i in range(T // _TCHUNK):
        sl = pl.ds(i * _TCHUNK, _TCHUNK)
        xb = xn_ref[sl, :]
        h = _matmul(xb, w1_ref[0]) + b1_ref[0]
        h = 0.5 * h * (1.0 + jax.lax.erf(h * _INV_SQRT2))
        y = _matmul(h, w2_ref[0]) + b2_ref[0]
        contrib = (RATIO * c[i * _TCHUNK:(i + 1) * _TCHUNK, :]) * y

        @pl.when(e == 0)
        def _first():
            o_ref[sl, :] = xb + contrib

        @pl.when(e > 0)
        def _rest():
            o_ref[sl, :] = o_ref[sl, :] + contrib


def _moe(xn, comb, e_w1, e_b1, e_w2, e_b2):
    return pl.pallas_call(
        _moe_body,
        grid=(E,),
        in_specs=[
            pl.BlockSpec((T, D), lambda e: (0, 0)),
            pl.BlockSpec((T, E), lambda e: (0, 0)),
            pl.BlockSpec((1, D, DFF), lambda e: (e, 0, 0)),
            pl.BlockSpec((1, 1, DFF), lambda e: (e, 0, 0)),
            pl.BlockSpec((1, DFF, D), lambda e: (e, 0, 0)),
            pl.BlockSpec((1, 1, D), lambda e: (e, 0, 0)),
        ],
        out_specs=pl.BlockSpec((T, D), lambda e: (0, 0)),
        out_shape=jax.ShapeDtypeStruct((T, D), jnp.float32),
    )(xn, comb, e_w1, e_b1.reshape(E, 1, DFF), e_w2, e_b2.reshape(E, 1, D))


def kernel(x, qkv_w0, qkv_b0, sa_w0, sa_b0, scale0, qkv_w1, qkv_b1, sa_w1,
           sa_b1, scale1, merger_w, merger_b, ln_g, ln_b, gate_w, gate_b,
           e_w1, e_b1, e_w2, e_b2):
    xf = x[0]

    qkv0, aw0 = _qkv_aware(xf, qkv_w0, qkv_b0, sa_w0, sa_b0, scale0)
    h0 = _flash_attention(qkv0, aw0)

    qkv1, aw1 = _qkv_aware(h0, qkv_w1, qkv_b1, sa_w1, sa_b1, scale1)
    prior_v = _matmul_bias(h0, qkv1[:, 2 * D:], jnp.zeros((D,), jnp.float32),
                           512)
    h1 = _flash_attention(qkv1, aw1, extra=prior_v)

    xn, comb = _merger_ln_gate(h0, h1, merger_w[:D], merger_w[D:], merger_b,
                               xf, ln_g, ln_b, gate_w, gate_b)
    out = _moe(xn, comb, e_w1, e_b1, e_w2, e_b2)
    return out.reshape(1, T, D)
